# baseline (speedup 1.0000x reference)
import functools

import jax
import jax.numpy as jnp
from jax import lax
from jax.experimental import pallas as pl
from jax.experimental.pallas import tpu as pltpu
from jax.experimental.pallas import tpu_sc as plsc

NUM_FEAT = 1000000
N_FIELD = 26
HIDDEN = 16
BATCH = 16384
N = BATCH * N_FIELD

NC = 2
NS = 16
NW = NC * NS

CHUNK = 1024
IDX_G = CHUNK // 128
BCHUNKS = BATCH // CHUNK
NCHUNKS = N_FIELD * BCHUNKS
CHUNKS_PER_W = NCHUNKS // NW


def _sc_body(x_hbm, cb_hbm, t_hbm, out_hbm, idx_v, res_v, cb_v,
             sem_g, sem_o):
    wid = lax.axis_index("s") * NC + lax.axis_index("c")
    pltpu.sync_copy(cb_hbm, cb_v)
    iota16 = lax.iota(jnp.int32, 16)

    def chunk_body(k, carry):
        chunk = wid * CHUNKS_PER_W + k
        f = chunk // BCHUNKS
        bc = chunk % BCHUNKS
        b0 = bc * CHUNK
        pltpu.sync_copy(x_hbm.at[f, bc], idx_v)
        cps = []
        for h in range(HIDDEN):
            plane = t_hbm.at[h]
            for g in range(IDX_G):
                cps.append(pltpu.async_copy(
                    plane.at[idx_v.at[g]],
                    res_v.at[pl.ds(h * CHUNK + g * 128, 128)], sem_g))
        for cp in cps:
            cp.wait()

        def vec_body(kk, c_):
            bvec = b0 + kk * 16 + iota16
            cvec = lax.shift_right_logical(bvec * N_FIELD + f, 14)
            for h in range(HIDDEN):
                hvec = jnp.broadcast_to(jnp.int32(h), (16,))
                cbv = plsc.load_gather(cb_v, [cvec, hvec])
                off = h * CHUNK + kk * 16
                v = res_v[pl.ds(off, 16)]
                res_v[pl.ds(off, 16)] = jnp.where(v != v, cbv, v)
            return c_

        lax.fori_loop(0, CHUNK // 16, vec_body, 0)
        ops = []
        for h in range(HIDDEN):
            ops.append(pltpu.async_copy(
                res_v.at[pl.ds(h * CHUNK, CHUNK)],
                out_hbm.at[f, h, pl.ds(b0, CHUNK)], sem_o))
        for op in ops:
            op.wait()
        return carry

    lax.fori_loop(0, CHUNKS_PER_W, chunk_body, 0)


@functools.partial(
    pl.kernel,
    out_type=jax.ShapeDtypeStruct((N_FIELD, HIDDEN, BATCH), jnp.float32),
    mesh=plsc.VectorSubcoreMesh(core_axis_name="c", subcore_axis_name="s",
                                num_cores=NC, num_subcores=NS),
    scratch_types=[
        pltpu.VMEM((IDX_G, 128), jnp.int32),
        pltpu.VMEM((HIDDEN * CHUNK,), jnp.float32),
        pltpu.VMEM((N_FIELD, HIDDEN), jnp.float32),
        pltpu.SemaphoreType.DMA,
        pltpu.SemaphoreType.DMA,
    ],
    compiler_params=pltpu.CompilerParams(use_tc_tiling_on_sc=False,
                                         needs_layout_passes=False),
)
def _sc_lookup(x_hbm, cb_hbm, t_hbm, out_hbm, idx_v, res_v, cb_v,
               sem_g, sem_o):
    _sc_body(x_hbm, cb_hbm, t_hbm, out_hbm, idx_v, res_v, cb_v,
             sem_g, sem_o)


def kernel(x, codebook, weight_sparse, keep_mask):
    x4 = x.T.reshape(N_FIELD, BCHUNKS, IDX_G, 128)
    combT = jnp.where(keep_mask.T, weight_sparse.T, jnp.float32(jnp.nan))
    out_cm = _sc_lookup(x4, codebook, combT)
    return jnp.transpose(out_cm, (2, 0, 1))


# speedup vs baseline: 2.5726x; 2.5726x over previous
import functools

import jax
import jax.numpy as jnp
from jax import lax
from jax.experimental import pallas as pl
from jax.experimental.pallas import tpu as pltpu
from jax.experimental.pallas import tpu_sc as plsc

NUM_FEAT = 1000000
N_FIELD = 26
HIDDEN = 16
BATCH = 16384
N = BATCH * N_FIELD

NC = 2
NS = 16
NW = NC * NS

CHUNK = 1024
IDX_G = CHUNK // 128
BCHUNKS = BATCH // CHUNK
NCHUNKS = N_FIELD * BCHUNKS
CHUNKS_PER_W = NCHUNKS // NW


def _sc_body(x_hbm, cb_hbm, t_hbm, out_hbm, idx_v, g_v, res_v, cb_v,
             sem_g, sem_o):
    wid = lax.axis_index("s") * NC + lax.axis_index("c")
    pltpu.sync_copy(cb_hbm, cb_v)
    lane_off = lax.iota(jnp.int32, 16) * CHUNK

    def chunk_body(k, carry):
        chunk = wid * CHUNKS_PER_W + k
        f = chunk // BCHUNKS
        bc = chunk % BCHUNKS
        b0 = bc * CHUNK
        pltpu.sync_copy(x_hbm.at[f, bc], idx_v)
        cps = []
        for g in range(IDX_G):
            cps.append(pltpu.async_copy(
                t_hbm.at[idx_v.at[g]], g_v.at[pl.ds(g * 128, 128)], sem_g))
        for cp in cps:
            cp.wait()

        def row_body(r, c_):
            i = (b0 + r) * N_FIELD + f
            crow = cb_v[i // BATCH]
            v = g_v[r]
            res = jnp.where(v != v, crow, v)
            plsc.store_scatter(res_v, [lane_off + r], res)
            return c_

        lax.fori_loop(0, CHUNK, row_body, 0, unroll=4)
        ops = []
        for h in range(HIDDEN):
            ops.append(pltpu.async_copy(
                res_v.at[pl.ds(h * CHUNK, CHUNK)],
                out_hbm.at[f, h, pl.ds(b0, CHUNK)], sem_o))
        for op in ops:
            op.wait()
        return carry

    lax.fori_loop(0, CHUNKS_PER_W, chunk_body, 0)


@functools.partial(
    pl.kernel,
    out_type=jax.ShapeDtypeStruct((N_FIELD, HIDDEN, BATCH), jnp.float32),
    mesh=plsc.VectorSubcoreMesh(core_axis_name="c", subcore_axis_name="s",
                                num_cores=NC, num_subcores=NS),
    scratch_types=[
        pltpu.VMEM((IDX_G, 128), jnp.int32),
        pltpu.VMEM((CHUNK, HIDDEN), jnp.float32),
        pltpu.VMEM((HIDDEN * CHUNK,), jnp.float32),
        pltpu.VMEM((N_FIELD, HIDDEN), jnp.float32),
        pltpu.SemaphoreType.DMA,
        pltpu.SemaphoreType.DMA,
    ],
    compiler_params=pltpu.CompilerParams(use_tc_tiling_on_sc=False,
                                         needs_layout_passes=False),
)
def _sc_lookup(x_hbm, cb_hbm, t_hbm, out_hbm, idx_v, g_v, res_v, cb_v,
               sem_g, sem_o):
    _sc_body(x_hbm, cb_hbm, t_hbm, out_hbm, idx_v, g_v, res_v, cb_v,
             sem_g, sem_o)


PK = 2048
_PACK_GRID = -(-NUM_FEAT // PK)


def _pack_body(m_ref, w_ref, out_ref):
    m = m_ref[...] != 0
    w = w_ref[...]
    comb = jnp.where(m, w, jnp.float32(jnp.nan))
    ct = comb.T
    c3 = ct.reshape(PK // 8, 8, HIDDEN)
    for r8 in range(8):
        out_ref[:, r8 * HIDDEN:(r8 + 1) * HIDDEN] = c3[:, r8, :]


_pack = pl.pallas_call(
    _pack_body,
    grid=(_PACK_GRID,),
    in_specs=[
        pl.BlockSpec((HIDDEN, PK), lambda i: (0, i)),
        pl.BlockSpec((HIDDEN, PK), lambda i: (0, i)),
    ],
    out_specs=pl.BlockSpec((PK * HIDDEN // 128, 128), lambda i: (i, 0)),
    out_shape=jax.ShapeDtypeStruct((NUM_FEAT * HIDDEN // 128, 128),
                                   jnp.float32),
)


def kernel(x, codebook, weight_sparse, keep_mask):
    x4 = x.T.reshape(N_FIELD, BCHUNKS, IDX_G, 128)
    mT8 = keep_mask.T.astype(jnp.int8)
    comb = _pack(mT8, weight_sparse.T).reshape(NUM_FEAT, HIDDEN)
    out_cm = _sc_lookup(x4, codebook, comb)
    return jnp.transpose(out_cm, (2, 0, 1))
